# Initial kernel scaffold; baseline (speedup 1.0000x reference)
import jax
import jax.numpy as jnp
from jax.experimental import pallas as pl


def kernel(x, pos_table):
    raise NotImplementedError("write your pallas kernel here")


# speedup vs baseline: 1.6652x; 1.6652x over previous
import jax
import jax.numpy as jnp
from jax.experimental import pallas as pl


BLK = 1024


def _add_block(x_ref, pos_ref, o_ref):
    o_ref[...] = x_ref[...] + pos_ref[...]


def kernel(x, pos_table):
    batch, seq, dim = x.shape
    ns = seq // BLK
    grid = (ns, batch)
    return pl.pallas_call(
        _add_block,
        grid=grid,
        in_specs=[
            pl.BlockSpec((1, BLK, dim), lambda i, j: (j, i, 0)),
            pl.BlockSpec((BLK, dim), lambda i, j: (i, 0)),
        ],
        out_specs=pl.BlockSpec((1, BLK, dim), lambda i, j: (j, i, 0)),
        out_shape=jax.ShapeDtypeStruct((batch, seq, dim), x.dtype),
    )(x, pos_table)


# speedup vs baseline: 1.7361x; 1.0426x over previous
import jax
import jax.numpy as jnp
from jax.experimental import pallas as pl


BLK = 2048


def _add_block(x_ref, pos_ref, o_ref):
    o_ref[...] = x_ref[...] + pos_ref[...]


def kernel(x, pos_table):
    batch, seq, dim = x.shape
    ns = seq // BLK
    grid = (ns, batch)
    return pl.pallas_call(
        _add_block,
        grid=grid,
        in_specs=[
            pl.BlockSpec((1, BLK, dim), lambda i, j: (j, i, 0)),
            pl.BlockSpec((BLK, dim), lambda i, j: (i, 0)),
        ],
        out_specs=pl.BlockSpec((1, BLK, dim), lambda i, j: (j, i, 0)),
        out_shape=jax.ShapeDtypeStruct((batch, seq, dim), x.dtype),
    )(x, pos_table)
